# Initial kernel scaffold; baseline (speedup 1.0000x reference)
import jax
import jax.numpy as jnp
from jax.experimental import pallas as pl


def kernel(features, edge_indices, edge_weights, W0, b0, A0, ab0, W1, b1, A1, ab1, Wp, bp, Wc1, bc1, Wc2, bc2):
    raise NotImplementedError("write your pallas kernel here")


# speedup vs baseline: 10.7518x; 10.7518x over previous
import functools

import jax
import jax.numpy as jnp
from jax import lax
from jax.experimental import pallas as pl
from jax.experimental.pallas import tpu as pltpu
from jax.experimental.pallas import tpu_sc as plsc

N = 10000
E = 320000
D = 128
HD = 64
W = 144
NW = 32
EPW = E // NW
B = 125
CHUNKS = EPW // B
STRIPE = N // 16


def _dot_t(x, w):
    return lax.dot_general(x, w, (((1,), (1,)), ((), ())),
                           preferred_element_type=jnp.float32)


def _pre_body(x_ref, w_ref, b_ref, a_ref, g_ref, h_ref, f_ref):
    _pre_body_vals(x_ref[...], w_ref, b_ref, a_ref, g_ref, h_ref, f_ref)


def _pre_body_vals(x, w_ref, b_ref, a_ref, g_ref, h_ref, f_ref):
    h = _dot_t(x, w_ref[...]) + b_ref[...]
    asrc = _dot_t(h, a_ref[:, :D])
    adst = _dot_t(h, a_ref[:, D:])
    e = jnp.exp(asrc - jnp.max(asrc, axis=0, keepdims=True))
    f_ref[...] = jnp.exp(adst - jnp.max(adst, axis=0, keepdims=True))
    h64 = h[:, :HD]
    g_ref[...] = jnp.concatenate(
        [h64 * e[:, 0:1], h64 * e[:, 1:2], e,
         jnp.zeros((h.shape[0], W - 2 * HD - 2), jnp.float32)], axis=1)
    h_ref[...] = h


def _finish_layer(p0, p1, h, f):
    acc = p0 + p1
    t = acc[:, 2 * HD:2 * HD + 2]
    z = jnp.sum(f * t, axis=0, keepdims=True)
    w = f / z
    agg = jnp.concatenate(
        [acc[:, :HD] * w[:, 0:1], acc[:, HD:2 * HD] * w[:, 1:2]], axis=1)
    return jax.nn.relu(agg + h)


def _fin_body(p0_ref, p1_ref, h_ref, f_ref, x_ref):
    x_ref[...] = _finish_layer(p0_ref[...], p1_ref[...], h_ref[...],
                               f_ref[...])


def _post_body(p0_ref, p1_ref, h_ref, f_ref, wc1_ref, bc1_ref, wc2_ref,
               bc2_ref, out_ref):
    x2 = _finish_layer(p0_ref[...], p1_ref[...], h_ref[...], f_ref[...])
    hc = jax.nn.relu(_dot_t(x2, wc1_ref[...]) + bc1_ref[...])
    out_ref[...] = _dot_t(hc, wc2_ref[...]) + bc2_ref[...]


_f32 = jnp.float32
_pre_call = pl.pallas_call(
    _pre_body,
    out_shape=(jax.ShapeDtypeStruct((N, W), _f32),
               jax.ShapeDtypeStruct((N, D), _f32),
               jax.ShapeDtypeStruct((N, 2), _f32)))

_fin_call = pl.pallas_call(
    _fin_body,
    out_shape=jax.ShapeDtypeStruct((N, D), _f32))

_post_call = pl.pallas_call(
    _post_body,
    out_shape=jax.ShapeDtypeStruct((N, 2), _f32))


def _sc_body(g_hbm, src_hbm, dst_hbm, zeros_hbm, out_hbm,
             src_v, dst_v, rows_v, acc_sh, sem):
    c = lax.axis_index("c")
    s = lax.axis_index("s")
    wid = c * 16 + s
    pltpu.sync_copy(src_hbm.at[wid], src_v)
    pltpu.sync_copy(dst_hbm.at[wid], dst_v)
    pltpu.sync_copy(zeros_hbm, acc_sh.at[pl.ds(s * STRIPE, STRIPE)])
    plsc.subcore_barrier()

    def body(i, _):
        pltpu.async_copy(g_hbm.at[src_v.at[i]], rows_v, sem).wait()
        pltpu.sync_copy(rows_v, acc_sh.at[dst_v.at[i]], add=True)
        return 0

    lax.fori_loop(0, CHUNKS, body, 0)
    plsc.subcore_barrier()
    pltpu.sync_copy(acc_sh.at[pl.ds(s * STRIPE, STRIPE)],
                    out_hbm.at[c, pl.ds(s * STRIPE, STRIPE)])


@functools.cache
def _sc_segsum_call():
    mesh = plsc.VectorSubcoreMesh(core_axis_name="c", subcore_axis_name="s",
                                  num_cores=2, num_subcores=16)
    return pl.kernel(
        _sc_body,
        out_type=jax.ShapeDtypeStruct((2, N, W), _f32),
        mesh=mesh,
        compiler_params=pltpu.CompilerParams(use_tc_tiling_on_sc=False),
        scratch_types=[
            pltpu.VMEM((CHUNKS, B), jnp.int32),
            pltpu.VMEM((CHUNKS, B), jnp.int32),
            pltpu.VMEM((B, W), _f32),
            pltpu.VMEM_SHARED((N, W), _f32),
            pltpu.SemaphoreType.DMA,
        ])


def kernel(features, edge_indices, edge_weights, W0, b0, A0, ab0,
           W1, b1, A1, ab1, Wp, bp, Wc1, bc1, Wc2, bc2):
    del edge_weights, ab0, ab1, Wp, bp
    edge_index = edge_indices[0]
    src3 = edge_index[0].reshape(NW, CHUNKS, B)
    dst3 = edge_index[1].reshape(NW, CHUNKS, B)
    zeros = jnp.zeros((STRIPE, W), _f32)

    sc = _sc_segsum_call()
    g0, h0, f0 = _pre_call(features, W0, b0.reshape(1, D), A0)
    p = sc(g0, src3, dst3, zeros)
    x1 = _fin_call(p[0], p[1], h0, f0)
    g1, h1, f1 = _pre_call(x1, W1, b1.reshape(1, D), A1)
    q = sc(g1, src3, dst3, zeros)
    logits = _post_call(q[0], q[1], h1, f1, Wc1, bc1.reshape(1, HD),
                        Wc2, bc2.reshape(1, 2))
    return logits
